# baseline (speedup 1.0000x reference)
import functools

import jax
import jax.numpy as jnp
from jax import lax
from jax.experimental import pallas as pl
from jax.experimental.pallas import tpu as pltpu
from jax.experimental.pallas import tpu_sc as plsc

CODEBOOK_SIZE = 8192
EMBEDDING_DIM = 256

R_BLK = 256
TILE_W = 2736


def _argmin_body(norm_ref, x_ref, cb_ref, out_ref):
    dot = lax.dot_general(
        x_ref[...].astype(jnp.bfloat16), cb_ref[...].astype(jnp.bfloat16),
        dimension_numbers=(((1,), (1,)), ((), ())),
        preferred_element_type=jnp.float32,
    )
    dist = norm_ref[...] - 2.0 * dot
    lane = lax.broadcasted_iota(jnp.int32, dist.shape, 1)
    inf = jnp.float32(jnp.inf)

    m = jnp.full((R_BLK, 1), inf, jnp.float32)
    idx = jnp.zeros((R_BLK, 1), jnp.int32)
    for t in range(3):
        lo, hi = t * TILE_W, min((t + 1) * TILE_W, CODEBOOK_SIZE)
        mask = (lane >= lo) & (lane < hi)
        dt = jnp.where(mask, dist, inf)
        tmin = jnp.min(dt, axis=1, keepdims=True)
        targ = jnp.min(jnp.where(dt == tmin, lane, CODEBOOK_SIZE),
                       axis=1, keepdims=True)
        upd = tmin < m
        idx = jnp.where(upd, targ, idx)
        m = jnp.where(upd, tmin, m).astype(jnp.bfloat16).astype(jnp.float32)
    out_ref[...] = idx


def _nearest_indices(flat, codebook, norm):
    n_rows = flat.shape[0]
    out = pl.pallas_call(
        _argmin_body,
        grid=(n_rows // R_BLK,),
        in_specs=[
            pl.BlockSpec((1, CODEBOOK_SIZE), lambda i: (0, 0)),
            pl.BlockSpec((R_BLK, EMBEDDING_DIM), lambda i: (i, 0)),
            pl.BlockSpec((CODEBOOK_SIZE, EMBEDDING_DIM), lambda i: (0, 0)),
        ],
        out_specs=pl.BlockSpec((R_BLK, 1), lambda i: (i, 0)),
        out_shape=jax.ShapeDtypeStruct((n_rows, 1), jnp.int32),
    )(norm.reshape(1, -1), flat, codebook)
    return out[:, 0]


def _sc_gather(codebook, idx):
    info = plsc.get_sparse_core_info()
    nw = info.num_cores * info.num_subcores
    b = idx.shape[0]
    b_per_w = b // nw
    chunk = min(b_per_w, 256)
    mesh = plsc.VectorSubcoreMesh(core_axis_name="c", subcore_axis_name="s")

    @functools.partial(
        pl.kernel, mesh=mesh,
        out_type=jax.ShapeDtypeStruct((b, EMBEDDING_DIM), jnp.float32),
        scratch_types=[
            pltpu.VMEM((chunk,), jnp.int32),
            pltpu.VMEM((chunk, EMBEDDING_DIM), jnp.float32),
            pltpu.SemaphoreType.DMA,
        ],
    )
    def gather_body(table_hbm, idx_hbm, out_hbm, idx_v, rows_v, sem):
        wid = lax.axis_index("s") * info.num_cores + lax.axis_index("c")
        base = wid * b_per_w
        for c in range(b_per_w // chunk):
            off = base + c * chunk
            pltpu.sync_copy(idx_hbm.at[pl.ds(off, chunk)], idx_v)
            pltpu.async_copy(table_hbm.at[idx_v], rows_v, sem).wait()
            pltpu.sync_copy(rows_v, out_hbm.at[pl.ds(off, chunk)])

    return gather_body(codebook, idx)


def kernel(input, codebook):
    batch_size = input.shape[0]
    embedding_dim = input.shape[1]
    spatial = input.shape[2:]
    z = input.reshape(batch_size, embedding_dim, -1)
    flat = jnp.transpose(z, (0, 2, 1)).reshape(-1, embedding_dim)
    norm = jnp.sum(codebook ** 2, axis=-1)

    indices = _nearest_indices(flat, codebook, norm)
    quantized = _sc_gather(codebook, indices)

    quantized = quantized.reshape(batch_size, -1, embedding_dim)
    quantized = jnp.transpose(quantized, (0, 2, 1))
    output = quantized.reshape(input.shape)
    indices_out = indices.reshape((batch_size,) + spatial).astype(jnp.int64)
    return output, indices_out


# speedup vs baseline: 1.2868x; 1.2868x over previous
import functools

import jax
import jax.numpy as jnp
from jax import lax
from jax.experimental import pallas as pl
from jax.experimental.pallas import tpu as pltpu
from jax.experimental.pallas import tpu_sc as plsc

CODEBOOK_SIZE = 8192
EMBEDDING_DIM = 256

R_BLK = 256
TILE_W = 2736


_SLICES = [
    (0, 2688), (2688, 2816), (2816, 5376), (5376, 5504), (5504, 8192)]
_SLICE_TILES = [
    (0,), (0, 1), (1,), (1, 2), (2,)]


def _argmin_body(norm_ref, x_ref, cb_ref, out_ref):
    dot = lax.dot_general(
        x_ref[...].astype(jnp.bfloat16), cb_ref[...].astype(jnp.bfloat16),
        dimension_numbers=(((1,), (1,)), ((), ())),
        preferred_element_type=jnp.float32,
    )
    dist = norm_ref[...] - 2.0 * dot
    inf = jnp.float32(jnp.inf)
    big = jnp.int32(CODEBOOK_SIZE)

    pieces = {0: [], 1: [], 2: []}
    for (lo, hi), tiles in zip(_SLICES, _SLICE_TILES):
        d = dist[:, lo:hi]
        it = lax.broadcasted_iota(jnp.int32, d.shape, 1) + lo
        if len(tiles) == 1:
            pieces[tiles[0]].append((d, it))
        else:
            bound = tiles[1] * TILE_W
            in_hi = it >= bound
            pieces[tiles[0]].append((jnp.where(in_hi, inf, d), it))
            pieces[tiles[1]].append((jnp.where(in_hi, d, inf), it))

    m = jnp.full((R_BLK, 1), inf, jnp.float32)
    idx = jnp.zeros((R_BLK, 1), jnp.int32)
    for t in range(3):
        tmin = jnp.min(
            jnp.concatenate([jnp.min(d, axis=1, keepdims=True)
                             for d, _ in pieces[t]], axis=1),
            axis=1, keepdims=True)
        targ = jnp.min(
            jnp.concatenate([jnp.min(jnp.where(d == tmin, it, big),
                                     axis=1, keepdims=True)
                             for d, it in pieces[t]], axis=1),
            axis=1, keepdims=True)
        upd = tmin < m
        idx = jnp.where(upd, targ, idx)
        m = jnp.where(upd, tmin, m).astype(jnp.bfloat16).astype(jnp.float32)
    out_ref[...] = idx


def _nearest_indices(flat, codebook, norm):
    n_rows = flat.shape[0]
    out = pl.pallas_call(
        _argmin_body,
        grid=(n_rows // R_BLK,),
        in_specs=[
            pl.BlockSpec((1, CODEBOOK_SIZE), lambda i: (0, 0)),
            pl.BlockSpec((R_BLK, EMBEDDING_DIM), lambda i: (i, 0)),
            pl.BlockSpec((CODEBOOK_SIZE, EMBEDDING_DIM), lambda i: (0, 0)),
        ],
        out_specs=pl.BlockSpec((R_BLK, 1), lambda i: (i, 0)),
        out_shape=jax.ShapeDtypeStruct((n_rows, 1), jnp.int32),
    )(norm.reshape(1, -1), flat, codebook)
    return out[:, 0]


def _sc_gather(codebook, idx):
    info = plsc.get_sparse_core_info()
    nw = info.num_cores * info.num_subcores
    b = idx.shape[0]
    b_per_w = b // nw
    chunk = min(b_per_w, 256)
    mesh = plsc.VectorSubcoreMesh(core_axis_name="c", subcore_axis_name="s")

    @functools.partial(
        pl.kernel, mesh=mesh,
        out_type=jax.ShapeDtypeStruct((b, EMBEDDING_DIM), jnp.float32),
        scratch_types=[
            pltpu.VMEM((chunk,), jnp.int32),
            pltpu.VMEM((chunk, EMBEDDING_DIM), jnp.float32),
            pltpu.SemaphoreType.DMA,
        ],
    )
    def gather_body(table_hbm, idx_hbm, out_hbm, idx_v, rows_v, sem):
        wid = lax.axis_index("s") * info.num_cores + lax.axis_index("c")
        base = wid * b_per_w
        for c in range(b_per_w // chunk):
            off = base + c * chunk
            pltpu.sync_copy(idx_hbm.at[pl.ds(off, chunk)], idx_v)
            pltpu.async_copy(table_hbm.at[idx_v], rows_v, sem).wait()
            pltpu.sync_copy(rows_v, out_hbm.at[pl.ds(off, chunk)])

    return gather_body(codebook, idx)


def kernel(input, codebook):
    batch_size = input.shape[0]
    embedding_dim = input.shape[1]
    spatial = input.shape[2:]
    z = input.reshape(batch_size, embedding_dim, -1)
    flat = jnp.transpose(z, (0, 2, 1)).reshape(-1, embedding_dim)
    norm = jnp.sum(codebook ** 2, axis=-1)

    indices = _nearest_indices(flat, codebook, norm)
    quantized = _sc_gather(codebook, indices)

    quantized = quantized.reshape(batch_size, -1, embedding_dim)
    quantized = jnp.transpose(quantized, (0, 2, 1))
    output = quantized.reshape(input.shape)
    indices_out = indices.reshape((batch_size,) + spatial).astype(jnp.int64)
    return output, indices_out


# speedup vs baseline: 1.3068x; 1.0156x over previous
import functools

import jax
import jax.numpy as jnp
from jax import lax
from jax.experimental import pallas as pl
from jax.experimental.pallas import tpu as pltpu
from jax.experimental.pallas import tpu_sc as plsc

CODEBOOK_SIZE = 8192
EMBEDDING_DIM = 256

R_BLK = 256
TILE_W = 2736


_SLICES = [
    (0, 2688), (2688, 2816), (2816, 5376), (5376, 5504), (5504, 8192)]
_SLICE_TILES = [
    (0,), (0, 1), (1,), (1, 2), (2,)]


def _argmin_body(norm_ref, x_ref, cb_ref, out_ref):
    dot2 = lax.dot_general(
        x_ref[...].astype(jnp.bfloat16), cb_ref[...].astype(jnp.bfloat16),
        dimension_numbers=(((1,), (1,)), ((), ())),
        preferred_element_type=jnp.float32,
    )
    dist = norm_ref[...] - dot2
    inf = jnp.float32(jnp.inf)
    big = jnp.int32(CODEBOOK_SIZE)

    pieces = {0: [], 1: [], 2: []}
    for (lo, hi), tiles in zip(_SLICES, _SLICE_TILES):
        d = dist[:, lo:hi]
        it = lax.broadcasted_iota(jnp.int32, d.shape, 1) + lo
        if len(tiles) == 1:
            pieces[tiles[0]].append((d, it))
        else:
            bound = tiles[1] * TILE_W
            in_hi = it >= bound
            pieces[tiles[0]].append((jnp.where(in_hi, inf, d), it))
            pieces[tiles[1]].append((jnp.where(in_hi, d, inf), it))

    m = jnp.full((R_BLK, 1), inf, jnp.float32)
    idx = jnp.zeros((R_BLK, 1), jnp.int32)
    for t in range(3):
        tmin = jnp.min(
            jnp.concatenate([jnp.min(d, axis=1, keepdims=True)
                             for d, _ in pieces[t]], axis=1),
            axis=1, keepdims=True)
        targ = jnp.min(
            jnp.concatenate([jnp.min(jnp.where(d == tmin, it, big),
                                     axis=1, keepdims=True)
                             for d, it in pieces[t]], axis=1),
            axis=1, keepdims=True)
        upd = tmin < m
        idx = jnp.where(upd, targ, idx)
        m = jnp.where(upd, tmin, m).astype(jnp.bfloat16).astype(jnp.float32)
    out_ref[...] = idx


def _nearest_indices(flat, codebook, norm):
    n_rows = flat.shape[0]
    out = pl.pallas_call(
        _argmin_body,
        grid=(n_rows // R_BLK,),
        in_specs=[
            pl.BlockSpec((1, CODEBOOK_SIZE), lambda i: (0, 0)),
            pl.BlockSpec((R_BLK, EMBEDDING_DIM), lambda i: (i, 0)),
            pl.BlockSpec((CODEBOOK_SIZE, EMBEDDING_DIM), lambda i: (0, 0)),
        ],
        out_specs=pl.BlockSpec((R_BLK, 1), lambda i: (i, 0)),
        out_shape=jax.ShapeDtypeStruct((n_rows, 1), jnp.int32),
    )(norm.reshape(1, -1), flat, codebook * 2.0)
    return out[:, 0]


def _sc_gather(codebook, idx):
    info = plsc.get_sparse_core_info()
    nw = info.num_cores * info.num_subcores
    b = idx.shape[0]
    b_per_w = b // nw
    chunk = min(b_per_w, 256)
    mesh = plsc.VectorSubcoreMesh(core_axis_name="c", subcore_axis_name="s")

    @functools.partial(
        pl.kernel, mesh=mesh,
        out_type=jax.ShapeDtypeStruct((b, EMBEDDING_DIM), jnp.float32),
        scratch_types=[
            pltpu.VMEM((chunk,), jnp.int32),
            pltpu.VMEM((chunk, EMBEDDING_DIM), jnp.float32),
            pltpu.SemaphoreType.DMA,
        ],
    )
    def gather_body(table_hbm, idx_hbm, out_hbm, idx_v, rows_v, sem):
        wid = lax.axis_index("s") * info.num_cores + lax.axis_index("c")
        base = wid * b_per_w
        for c in range(b_per_w // chunk):
            off = base + c * chunk
            pltpu.sync_copy(idx_hbm.at[pl.ds(off, chunk)], idx_v)
            pltpu.async_copy(table_hbm.at[idx_v], rows_v, sem).wait()
            pltpu.sync_copy(rows_v, out_hbm.at[pl.ds(off, chunk)])

    return gather_body(codebook, idx)


def kernel(input, codebook):
    batch_size = input.shape[0]
    embedding_dim = input.shape[1]
    spatial = input.shape[2:]
    z = input.reshape(batch_size, embedding_dim, -1)
    flat = jnp.transpose(z, (0, 2, 1)).reshape(-1, embedding_dim)
    norm = jnp.sum(codebook ** 2, axis=-1)

    indices = _nearest_indices(flat, codebook, norm)
    quantized = _sc_gather(codebook, indices)

    quantized = quantized.reshape(batch_size, -1, embedding_dim)
    quantized = jnp.transpose(quantized, (0, 2, 1))
    output = quantized.reshape(input.shape)
    indices_out = indices.reshape((batch_size,) + spatial).astype(jnp.int64)
    return output, indices_out


# speedup vs baseline: 1.3509x; 1.0338x over previous
import functools

import jax
import jax.numpy as jnp
from jax import lax
from jax.experimental import pallas as pl
from jax.experimental.pallas import tpu as pltpu
from jax.experimental.pallas import tpu_sc as plsc

CODEBOOK_SIZE = 8192
EMBEDDING_DIM = 256

R_BLK = 256
TILE_W = 2736


_SLICES = [
    (0, 2688), (2688, 2816), (2816, 5376), (5376, 5504), (5504, 8192)]
_SLICE_TILES = [
    (0,), (0, 1), (1,), (1, 2), (2,)]


def _argmin_body(norm_ref, x_ref, cb_ref, out_ref):
    dot2 = lax.dot_general(
        x_ref[...].astype(jnp.bfloat16), cb_ref[...],
        dimension_numbers=(((1,), (1,)), ((), ())),
        preferred_element_type=jnp.float32,
    )
    dist = norm_ref[...] - dot2
    inf = jnp.float32(jnp.inf)
    big = jnp.int32(CODEBOOK_SIZE)

    pieces = {0: [], 1: [], 2: []}
    for (lo, hi), tiles in zip(_SLICES, _SLICE_TILES):
        d = dist[:, lo:hi]
        it = lax.broadcasted_iota(jnp.int32, d.shape, 1) + lo
        if len(tiles) == 1:
            pieces[tiles[0]].append((d, it))
        else:
            bound = tiles[1] * TILE_W
            in_hi = it >= bound
            pieces[tiles[0]].append((jnp.where(in_hi, inf, d), it))
            pieces[tiles[1]].append((jnp.where(in_hi, d, inf), it))

    m = jnp.full((R_BLK, 1), inf, jnp.float32)
    idx = jnp.zeros((R_BLK, 1), jnp.int32)
    for t in range(3):
        tmin = jnp.min(
            jnp.concatenate([jnp.min(d, axis=1, keepdims=True)
                             for d, _ in pieces[t]], axis=1),
            axis=1, keepdims=True)
        targ = jnp.min(
            jnp.concatenate([jnp.min(jnp.where(d == tmin, it, big),
                                     axis=1, keepdims=True)
                             for d, it in pieces[t]], axis=1),
            axis=1, keepdims=True)
        upd = tmin < m
        idx = jnp.where(upd, targ, idx)
        m = jnp.where(upd, tmin, m).astype(jnp.bfloat16).astype(jnp.float32)
    out_ref[...] = idx


def _nearest_indices(flat, codebook, norm):
    n_rows = flat.shape[0]
    out = pl.pallas_call(
        _argmin_body,
        grid=(n_rows // R_BLK,),
        in_specs=[
            pl.BlockSpec((1, CODEBOOK_SIZE), lambda i: (0, 0)),
            pl.BlockSpec((R_BLK, EMBEDDING_DIM), lambda i: (i, 0)),
            pl.BlockSpec((CODEBOOK_SIZE, EMBEDDING_DIM), lambda i: (0, 0)),
        ],
        out_specs=pl.BlockSpec((R_BLK, 1), lambda i: (i, 0)),
        out_shape=jax.ShapeDtypeStruct((n_rows, 1), jnp.int32),
    )(norm.reshape(1, -1), flat, (codebook * 2.0).astype(jnp.bfloat16))
    return out[:, 0]


def _sc_gather(codebook, idx):
    info = plsc.get_sparse_core_info()
    nw = info.num_cores * info.num_subcores
    b = idx.shape[0]
    b_per_w = b // nw
    chunk = min(b_per_w, 256)
    mesh = plsc.VectorSubcoreMesh(core_axis_name="c", subcore_axis_name="s")

    @functools.partial(
        pl.kernel, mesh=mesh,
        out_type=jax.ShapeDtypeStruct((b, EMBEDDING_DIM), jnp.float32),
        scratch_types=[
            pltpu.VMEM((chunk,), jnp.int32),
            pltpu.VMEM((chunk, EMBEDDING_DIM), jnp.float32),
            pltpu.SemaphoreType.DMA,
        ],
    )
    def gather_body(table_hbm, idx_hbm, out_hbm, idx_v, rows_v, sem):
        wid = lax.axis_index("s") * info.num_cores + lax.axis_index("c")
        base = wid * b_per_w
        for c in range(b_per_w // chunk):
            off = base + c * chunk
            pltpu.sync_copy(idx_hbm.at[pl.ds(off, chunk)], idx_v)
            pltpu.async_copy(table_hbm.at[idx_v], rows_v, sem).wait()
            pltpu.sync_copy(rows_v, out_hbm.at[pl.ds(off, chunk)])

    return gather_body(codebook, idx)


def kernel(input, codebook):
    batch_size = input.shape[0]
    embedding_dim = input.shape[1]
    spatial = input.shape[2:]
    z = input.reshape(batch_size, embedding_dim, -1)
    flat = jnp.transpose(z, (0, 2, 1)).reshape(-1, embedding_dim)
    norm = jnp.sum(codebook ** 2, axis=-1)

    indices = _nearest_indices(flat, codebook, norm)
    quantized = _sc_gather(codebook, indices)

    quantized = quantized.reshape(batch_size, -1, embedding_dim)
    quantized = jnp.transpose(quantized, (0, 2, 1))
    output = quantized.reshape(input.shape)
    indices_out = indices.reshape((batch_size,) + spatial).astype(jnp.int64)
    return output, indices_out


# speedup vs baseline: 1.4308x; 1.0591x over previous
import functools

import jax
import jax.numpy as jnp
from jax import lax
from jax.experimental import pallas as pl
from jax.experimental.pallas import tpu as pltpu
from jax.experimental.pallas import tpu_sc as plsc

CODEBOOK_SIZE = 8192
EMBEDDING_DIM = 256

R_BLK = 256
TILE_W = 2736


_SLICES = [
    (0, 2688), (2688, 2816), (2816, 5376), (5376, 5504), (5504, 8192)]
_SLICE_TILES = [
    (0,), (0, 1), (1,), (1, 2), (2,)]


def _argmin_body(norm_ref, x_ref, cb_ref, out_ref):
    dot2 = lax.dot_general(
        x_ref[...].astype(jnp.bfloat16), cb_ref[...],
        dimension_numbers=(((1,), (1,)), ((), ())),
        preferred_element_type=jnp.float32,
    )
    dist = norm_ref[...] - dot2
    inf = jnp.float32(jnp.inf)
    big = jnp.int32(CODEBOOK_SIZE)

    pieces = {0: [], 1: [], 2: []}
    for (lo, hi), tiles in zip(_SLICES, _SLICE_TILES):
        d = dist[:, lo:hi]
        if len(tiles) == 1:
            pieces[tiles[0]].append((d, lo))
        else:
            it = lax.broadcasted_iota(jnp.int32, d.shape, 1)
            in_hi = it >= (tiles[1] * TILE_W - lo)
            pieces[tiles[0]].append((jnp.where(in_hi, inf, d), lo))
            pieces[tiles[1]].append((jnp.where(in_hi, d, inf), lo))

    m = jnp.full((R_BLK, 1), inf, jnp.float32)
    idx = jnp.zeros((R_BLK, 1), jnp.int32)
    for t in range(3):
        mins = [jnp.min(d, axis=1, keepdims=True) for d, _ in pieces[t]]
        tmin = mins[0]
        for piece_min in mins[1:]:
            tmin = jnp.minimum(tmin, piece_min)
        targ = None
        for (d, lo), piece_min in zip(pieces[t], mins):
            it = lax.broadcasted_iota(jnp.int32, d.shape, 1)
            cand = jnp.min(jnp.where(d == tmin, it, big),
                           axis=1, keepdims=True) + lo
            cand = jnp.where(piece_min == tmin, cand, big)
            targ = cand if targ is None else jnp.minimum(targ, cand)
        upd = tmin < m
        idx = jnp.where(upd, targ, idx)
        m = jnp.where(upd, tmin, m).astype(jnp.bfloat16).astype(jnp.float32)
    out_ref[...] = idx


def _nearest_indices(flat, codebook, norm):
    n_rows = flat.shape[0]
    out = pl.pallas_call(
        _argmin_body,
        grid=(n_rows // R_BLK,),
        in_specs=[
            pl.BlockSpec((1, CODEBOOK_SIZE), lambda i: (0, 0)),
            pl.BlockSpec((R_BLK, EMBEDDING_DIM), lambda i: (i, 0)),
            pl.BlockSpec((CODEBOOK_SIZE, EMBEDDING_DIM), lambda i: (0, 0)),
        ],
        out_specs=pl.BlockSpec((R_BLK, 1), lambda i: (i, 0)),
        out_shape=jax.ShapeDtypeStruct((n_rows, 1), jnp.int32),
    )(norm.reshape(1, -1), flat, (codebook * 2.0).astype(jnp.bfloat16))
    return out[:, 0]


def _sc_gather(codebook, idx):
    info = plsc.get_sparse_core_info()
    nw = info.num_cores * info.num_subcores
    b = idx.shape[0]
    b_per_w = b // nw
    chunk = min(b_per_w, 256)
    mesh = plsc.VectorSubcoreMesh(core_axis_name="c", subcore_axis_name="s")

    @functools.partial(
        pl.kernel, mesh=mesh,
        out_type=jax.ShapeDtypeStruct((b, EMBEDDING_DIM), jnp.float32),
        scratch_types=[
            pltpu.VMEM((chunk,), jnp.int32),
            pltpu.VMEM((chunk, EMBEDDING_DIM), jnp.float32),
            pltpu.SemaphoreType.DMA,
        ],
    )
    def gather_body(table_hbm, idx_hbm, out_hbm, idx_v, rows_v, sem):
        wid = lax.axis_index("s") * info.num_cores + lax.axis_index("c")
        base = wid * b_per_w
        for c in range(b_per_w // chunk):
            off = base + c * chunk
            pltpu.sync_copy(idx_hbm.at[pl.ds(off, chunk)], idx_v)
            pltpu.async_copy(table_hbm.at[idx_v], rows_v, sem).wait()
            pltpu.sync_copy(rows_v, out_hbm.at[pl.ds(off, chunk)])

    return gather_body(codebook, idx)


def kernel(input, codebook):
    batch_size = input.shape[0]
    embedding_dim = input.shape[1]
    spatial = input.shape[2:]
    z = input.reshape(batch_size, embedding_dim, -1)
    flat = jnp.transpose(z, (0, 2, 1)).reshape(-1, embedding_dim)
    norm = jnp.sum(codebook ** 2, axis=-1)

    indices = _nearest_indices(flat, codebook, norm)
    quantized = _sc_gather(codebook, indices)

    quantized = quantized.reshape(batch_size, -1, embedding_dim)
    quantized = jnp.transpose(quantized, (0, 2, 1))
    output = quantized.reshape(input.shape)
    indices_out = indices.reshape((batch_size,) + spatial).astype(jnp.int64)
    return output, indices_out
